# baseline (speedup 1.0000x reference)
import functools

import numpy as np

import jax
import jax.numpy as jnp
from jax import lax
from jax.experimental import pallas as pl
from jax.experimental.pallas import tpu as pltpu
from jax.experimental.pallas import tpu_sc as plsc

B = 4096
NUM_NUM = 13
NUM_CAT = 26
VOCAB = 100000
D = 64

NC = 2
NS = 16
NW = NC * NS
TOT_ROWS = B * NUM_CAT
ROWS_PER_W = TOT_ROWS // NW
N_GATHERS = ROWS_PER_W // 128
G_PER_CHUNK = 2
N_CHUNKS = N_GATHERS // G_PER_CHUNK
CHUNK_ROWS = G_PER_CHUNK * 128

PAIR_VOCAB = NUM_CAT * VOCAB // 2


def _sc_gather_body(tables_hbm, idx_hbm, out_hbm, idx_v, rows_v, sem):
    wid = lax.axis_index("s") * NC + lax.axis_index("c")
    base = wid * ROWS_PER_W
    pltpu.sync_copy(idx_hbm.at[wid], idx_v)
    for c in range(N_CHUNKS):
        copies = []
        for j in range(G_PER_CHUNK):
            copies.append(pltpu.async_copy(
                tables_hbm.at[idx_v.at[pl.ds((c * G_PER_CHUNK + j) * 128, 128)]],
                rows_v.at[pl.ds(j * 128, 128)],
                sem))
        for cp in copies:
            cp.wait()
        pltpu.sync_copy(
            rows_v, out_hbm.at[pl.ds(base + c * CHUNK_ROWS, CHUNK_ROWS)])


@functools.partial(jax.jit, static_argnums=())
def _sc_gather(tables2, idx2):
    mesh = plsc.VectorSubcoreMesh(
        core_axis_name="c", subcore_axis_name="s", num_cores=NC,
        num_subcores=NS)
    k = pl.kernel(
        _sc_gather_body,
        out_type=jax.ShapeDtypeStruct((TOT_ROWS, 2 * D), jnp.float32),
        mesh=mesh,
        scratch_types=[
            pltpu.VMEM((ROWS_PER_W,), jnp.int32),
            pltpu.VMEM((CHUNK_ROWS, 2 * D), jnp.float32),
            pltpu.SemaphoreType.DMA,
        ],
    )
    return k(tables2, idx2)


BB = 512
N_FEAT = NUM_CAT + 1
N_TRI = N_FEAT * (N_FEAT - 1) // 2
T_COLS = D + N_TRI + 1


def _tc_body(num_ref, emb_ref, par_ref, bw1, bb1, bw2, bb2, bw3, bb3,
             tw1b, tw1e, tb1, tw2, tb2, tw3, tb3, tw4, tb4, tw5, tb5, out_ref):
    f32 = jnp.float32
    x = num_ref[...]
    h = jax.nn.relu(jnp.dot(x, bw1[...], preferred_element_type=f32) + bb1[...])
    h = jax.nn.relu(jnp.dot(h, bw2[...], preferred_element_type=f32) + bb2[...])
    bottom = jax.nn.relu(
        jnp.dot(h, bw3[...], preferred_element_type=f32) + bb3[...])

    pair = emb_ref[...]
    odd = par_ref[...] == 1
    emb = jnp.where(odd, pair[:, :, D:], pair[:, :, :D])
    g = jnp.concatenate([bottom[:, None, :], emb], axis=1)

    inter = lax.dot_general(g, g, (((2,), (2,)), ((0,), (0,))),
                            preferred_element_type=f32)
    r = inter.reshape(BB, N_FEAT * N_FEAT)

    t = jax.nn.relu(jnp.dot(bottom, tw1b[...], preferred_element_type=f32)
                    + jnp.dot(r, tw1e[...], preferred_element_type=f32)
                    + tb1[...])
    t = jax.nn.relu(jnp.dot(t, tw2[...], preferred_element_type=f32) + tb2[...])
    t = jax.nn.relu(jnp.dot(t, tw3[...], preferred_element_type=f32) + tb3[...])
    t = jax.nn.relu(jnp.dot(t, tw4[...], preferred_element_type=f32) + tb4[...])
    out_ref[...] = jax.nn.sigmoid(
        jnp.dot(t, tw5[...], preferred_element_type=f32) + tb5[...])


def _full(shape):
    return pl.BlockSpec(shape, lambda i: (0,) * len(shape))


def _tc_fused(numerical, emb3, parity, bw1, bb1, bw2, bb2, bw3, bb3,
              tw1b, tw1e, tb1, tw2, tb2, tw3, tb3, tw4, tb4, tw5, tb5):
    grid = (B // BB,)
    in_specs = [
        pl.BlockSpec((BB, NUM_NUM), lambda i: (i, 0)),
        pl.BlockSpec((BB, NUM_CAT, 2 * D), lambda i: (i, 0, 0)),
        pl.BlockSpec((BB, NUM_CAT, 1), lambda i: (i, 0, 0)),
    ]
    weights = (bw1, bb1, bw2, bb2, bw3, bb3,
               tw1b, tw1e, tb1, tw2, tb2, tw3, tb3, tw4, tb4, tw5, tb5)
    for w in weights:
        in_specs.append(_full(w.shape))
    return pl.pallas_call(
        _tc_body,
        grid=grid,
        in_specs=in_specs,
        out_specs=pl.BlockSpec((BB, 1), lambda i: (i, 0)),
        out_shape=jax.ShapeDtypeStruct((B, 1), jnp.float32),
        compiler_params=pltpu.CompilerParams(
            dimension_semantics=("arbitrary",)),
    )(numerical, emb3, parity, *weights)


def kernel(numerical_input, categorical_inputs, tables,
           bw1, bb1, bw2, bb2, bw3, bb3,
           tw1, tb1, tw2, tb2, tw3, tb3, tw4, tb4, tw5, tb5):
    offsets = jnp.arange(NUM_CAT, dtype=jnp.int32) * VOCAB
    flat_idx = (categorical_inputs.astype(jnp.int32)
                + offsets[None, :])
    pair_idx = (flat_idx >> 1).reshape(NW, ROWS_PER_W)
    parity = (flat_idx & 1)[:, :, None]
    tables2 = tables.reshape(PAIR_VOCAB, 2 * D)
    emb_flat = _sc_gather(tables2, pair_idx)
    emb3 = emb_flat.reshape(B, NUM_CAT, 2 * D)
    biases = [b.reshape(1, -1) for b in (bb1, bb2, bb3, tb1, tb2, tb3, tb4, tb5)]
    rows, cols = np.tril_indices(N_FEAT, -1)
    pos = jnp.asarray(rows * N_FEAT + cols, dtype=jnp.int32)
    tw1b = tw1[:D]
    tw1e = jnp.zeros((N_FEAT * N_FEAT, tw1.shape[1]), jnp.float32)
    tw1e = tw1e.at[pos].set(tw1[D:D + N_TRI])
    return _tc_fused(numerical_input, emb3, parity,
                     bw1, biases[0], bw2, biases[1], bw3, biases[2],
                     tw1b, tw1e, biases[3], tw2, biases[4], tw3, biases[5],
                     tw4, biases[6], tw5, biases[7])


# speedup vs baseline: 1.0584x; 1.0584x over previous
import functools

import numpy as np

import jax
import jax.numpy as jnp
from jax import lax
from jax.experimental import pallas as pl
from jax.experimental.pallas import tpu as pltpu
from jax.experimental.pallas import tpu_sc as plsc

B = 4096
NUM_NUM = 13
NUM_CAT = 26
VOCAB = 100000
D = 64

NC = 2
NS = 16
NW = NC * NS
TOT_ROWS = B * NUM_CAT
ROWS_PER_W = TOT_ROWS // NW
N_GATHERS = ROWS_PER_W // 128
G_PER_CHUNK = 13
N_CHUNKS = N_GATHERS // G_PER_CHUNK
CHUNK_ROWS = G_PER_CHUNK * 128


def _sc_gather_body(tables_hbm, idx_hbm, out_hbm, idx_v, rows_v, sem):
    wid = lax.axis_index("s") * NC + lax.axis_index("c")
    base = wid * ROWS_PER_W
    pltpu.sync_copy(idx_hbm.at[wid], idx_v)
    for c in range(N_CHUNKS):
        copies = []
        for j in range(G_PER_CHUNK):
            copies.append(pltpu.async_copy(
                tables_hbm.at[idx_v.at[pl.ds((c * G_PER_CHUNK + j) * 128, 128)]],
                rows_v.at[pl.ds(j * 128, 128)],
                sem))
        for cp in copies:
            cp.wait()
        pltpu.sync_copy(
            rows_v, out_hbm.at[pl.ds(base + c * CHUNK_ROWS, CHUNK_ROWS)])


@functools.partial(jax.jit, static_argnums=())
def _sc_gather(tables, idx2):
    mesh = plsc.VectorSubcoreMesh(
        core_axis_name="c", subcore_axis_name="s", num_cores=NC,
        num_subcores=NS)
    k = pl.kernel(
        _sc_gather_body,
        out_type=jax.ShapeDtypeStruct((TOT_ROWS, D), jnp.float32),
        mesh=mesh,
        scratch_types=[
            pltpu.VMEM((ROWS_PER_W,), jnp.int32),
            pltpu.VMEM((CHUNK_ROWS, D), jnp.float32),
            pltpu.SemaphoreType.DMA,
        ],
        compiler_params=pltpu.CompilerParams(use_tc_tiling_on_sc=False),
    )
    return k(tables, idx2)


BB = 512
N_FEAT = NUM_CAT + 1
N_TRI = N_FEAT * (N_FEAT - 1) // 2
T_COLS = D + N_TRI + 1


def _tc_body(num_ref, emb_ref, bw1, bb1, bw2, bb2, bw3, bb3,
             tw1b, tw1e, tb1, tw2, tb2, tw3, tb3, tw4, tb4, tw5, tb5, out_ref):
    f32 = jnp.float32
    x = num_ref[...]
    h = jax.nn.relu(jnp.dot(x, bw1[...], preferred_element_type=f32) + bb1[...])
    h = jax.nn.relu(jnp.dot(h, bw2[...], preferred_element_type=f32) + bb2[...])
    bottom = jax.nn.relu(
        jnp.dot(h, bw3[...], preferred_element_type=f32) + bb3[...])

    emb = emb_ref[...]
    g = jnp.concatenate([bottom[:, None, :], emb], axis=1)

    inter = lax.dot_general(g, g, (((2,), (2,)), ((0,), (0,))),
                            preferred_element_type=f32)
    r = inter.reshape(BB, N_FEAT * N_FEAT)

    t = jax.nn.relu(jnp.dot(bottom, tw1b[...], preferred_element_type=f32)
                    + jnp.dot(r, tw1e[...], preferred_element_type=f32)
                    + tb1[...])
    t = jax.nn.relu(jnp.dot(t, tw2[...], preferred_element_type=f32) + tb2[...])
    t = jax.nn.relu(jnp.dot(t, tw3[...], preferred_element_type=f32) + tb3[...])
    t = jax.nn.relu(jnp.dot(t, tw4[...], preferred_element_type=f32) + tb4[...])
    out_ref[...] = jax.nn.sigmoid(
        jnp.dot(t, tw5[...], preferred_element_type=f32) + tb5[...])


def _full(shape):
    return pl.BlockSpec(shape, lambda i: (0,) * len(shape))


def _tc_fused(numerical, emb3, bw1, bb1, bw2, bb2, bw3, bb3,
              tw1b, tw1e, tb1, tw2, tb2, tw3, tb3, tw4, tb4, tw5, tb5):
    grid = (B // BB,)
    in_specs = [
        pl.BlockSpec((BB, NUM_NUM), lambda i: (i, 0)),
        pl.BlockSpec((BB, NUM_CAT, D), lambda i: (i, 0, 0)),
    ]
    weights = (bw1, bb1, bw2, bb2, bw3, bb3,
               tw1b, tw1e, tb1, tw2, tb2, tw3, tb3, tw4, tb4, tw5, tb5)
    for w in weights:
        in_specs.append(_full(w.shape))
    return pl.pallas_call(
        _tc_body,
        grid=grid,
        in_specs=in_specs,
        out_specs=pl.BlockSpec((BB, 1), lambda i: (i, 0)),
        out_shape=jax.ShapeDtypeStruct((B, 1), jnp.float32),
        compiler_params=pltpu.CompilerParams(
            dimension_semantics=("arbitrary",)),
    )(numerical, emb3, *weights)


def kernel(numerical_input, categorical_inputs, tables,
           bw1, bb1, bw2, bb2, bw3, bb3,
           tw1, tb1, tw2, tb2, tw3, tb3, tw4, tb4, tw5, tb5):
    offsets = jnp.arange(NUM_CAT, dtype=jnp.int32) * VOCAB
    flat_idx = (categorical_inputs.astype(jnp.int32)
                + offsets[None, :])
    idx2 = flat_idx.reshape(NW, ROWS_PER_W)
    emb_flat = _sc_gather(tables, idx2)
    emb3 = emb_flat.reshape(B, NUM_CAT, D)
    biases = [b.reshape(1, -1) for b in (bb1, bb2, bb3, tb1, tb2, tb3, tb4, tb5)]
    rows, cols = np.tril_indices(N_FEAT, -1)
    pos = jnp.asarray(rows * N_FEAT + cols, dtype=jnp.int32)
    tw1b = tw1[:D]
    tw1e = jnp.zeros((N_FEAT * N_FEAT, tw1.shape[1]), jnp.float32)
    tw1e = tw1e.at[pos].set(tw1[D:D + N_TRI])
    return _tc_fused(numerical_input, emb3,
                     bw1, biases[0], bw2, biases[1], bw3, biases[2],
                     tw1b, tw1e, biases[3], tw2, biases[4], tw3, biases[5],
                     tw4, biases[6], tw5, biases[7])
